# baseline (speedup 1.0000x reference)
import jax
import jax.numpy as jnp
from jax import lax
from jax.experimental import pallas as pl
from jax.experimental.pallas import tpu as pltpu
from jax.experimental.pallas import tpu_sc as plsc

NUM_CORES = 2
NUM_SUBCORES = 16
NUM_WORKERS = NUM_CORES * NUM_SUBCORES
LANES = 16

B, L, D = 4096, 200, 64
CHUNK = B // NUM_WORKERS
NBUF = 5
AHEAD = 3
NG = D // (2 * LANES)


def _sc_body(tok_hbm, ids_hbm, pos_hbm, out_hbm,
             idx_all, pos_v,
             gb0, gb1, gb2, gb3, gb4,
             ob0, ob1, ob2, ob3, ob4,
             g0, g1, g2, g3, g4, s0, s1, s2, s3, s4):
    gbufs = (gb0, gb1, gb2, gb3, gb4)
    obufs = (ob0, ob1, ob2, ob3, ob4)
    gsems = (g0, g1, g2, g3, g4)
    ssems = (s0, s1, s2, s3, s4)

    wid = lax.axis_index("s") * NUM_CORES + lax.axis_index("c")
    b0 = wid * CHUNK

    pltpu.sync_copy(ids_hbm.at[:, pl.ds(b0, CHUNK)], idx_all)
    pltpu.sync_copy(pos_hbm, pos_v)

    for k in range(AHEAD):
        pltpu.async_copy(tok_hbm.at[idx_all.at[k]], gbufs[k], gsems[k])

    def process(l, k):
        gbuf, obuf, gsem, ssem = gbufs[k], obufs[k], gsems[k], ssems[k]
        pltpu.make_async_copy(tok_hbm.at[pl.ds(0, CHUNK)], gbuf, gsem).wait()

        pvecs = [(pos_v[l, pl.ds(32 * j, LANES)],
                  pos_v[l, pl.ds(32 * j + LANES, LANES)]) for j in range(NG)]

        def radd(r, carry):
            for j in range(NG):
                v = plsc.bitcast(gbuf[r, pl.ds(32 * j, 2 * LANES)], jnp.int32)
                a = plsc.bitcast(v << 16, jnp.float32)
                b = plsc.bitcast(v & jnp.int32(-65536), jnp.float32)
                obuf[r, pl.ds(32 * j, LANES)] = a + pvecs[j][0]
                obuf[r, pl.ds(32 * j + LANES, LANES)] = b + pvecs[j][1]
            return carry

        lax.fori_loop(0, CHUNK, radd, 0, unroll=8)

        pltpu.async_copy(obuf, out_hbm.at[pl.ds(b0, CHUNK), l, pl.ds(0, D)],
                         ssem)

        ka = (k + AHEAD) % NBUF
        la = l + AHEAD

        @pl.when(la < L)
        def _():
            @pl.when(la >= NBUF)
            def _():
                pltpu.make_async_copy(
                    obufs[ka], out_hbm.at[pl.ds(b0, CHUNK), 0, pl.ds(0, D)],
                    ssems[ka]).wait()
            pltpu.async_copy(tok_hbm.at[idx_all.at[la]], gbufs[ka], gsems[ka])

    def ring_lap(i, carry):
        l = NBUF * i
        for k in range(NBUF):
            process(l + k, k)
        return carry

    lax.fori_loop(0, L // NBUF, ring_lap, 0)

    for k in range(NBUF):
        pltpu.make_async_copy(obufs[k],
                              out_hbm.at[pl.ds(b0, CHUNK), 0, pl.ds(0, D)],
                              ssems[k]).wait()


def _make_sc_kernel():
    mesh = plsc.VectorSubcoreMesh(core_axis_name="c", subcore_axis_name="s")
    return pl.kernel(
        _sc_body,
        out_type=jax.ShapeDtypeStruct((B, L, 2 * D), jnp.float32),
        mesh=mesh,
        scratch_types=[
            pltpu.VMEM((L, CHUNK), jnp.int32),
            pltpu.VMEM((L, D), jnp.float32),
        ] + [pltpu.VMEM((CHUNK, D), jnp.bfloat16)] * NBUF
          + [pltpu.VMEM((CHUNK, D), jnp.float32)] * NBUF
          + [pltpu.SemaphoreType.DMA] * (2 * NBUF),
        compiler_params=pltpu.CompilerParams(use_tc_tiling_on_sc=False,
                                             needs_layout_passes=False),
    )


def _kernel_impl(ids, tok, pos):
    ids_t = ids.astype(jnp.int32).T
    pos2d = pos.reshape(L, D).astype(jnp.float32)
    perm = [0] * D
    for g in range(NG):
        for i in range(LANES):
            perm[32 * g + 2 * i] = 32 * g + i
            perm[32 * g + 2 * i + 1] = 32 * g + LANES + i
    tok_bf = tok[:, jnp.array(perm)].astype(jnp.bfloat16)
    tok_a = lax.optimization_barrier(tok_bf.reshape(500000, 128))
    tok_b = tok_a.reshape(1000000, 64)
    out_p = _make_sc_kernel()(tok_b, ids_t, pos2d)
    return out_p[:, :, :D]


kernel = jax.jit(_kernel_impl)


# speedup vs baseline: 1.9929x; 1.9929x over previous
import jax
import jax.numpy as jnp
from jax import lax
from jax.experimental import pallas as pl
from jax.experimental.pallas import tpu as pltpu
from jax.experimental.pallas import tpu_sc as plsc

NUM_CORES = 2
NUM_SUBCORES = 16
NUM_WORKERS = NUM_CORES * NUM_SUBCORES
LANES = 16

B, L, D = 4096, 200, 64
CHUNK = B // NUM_WORKERS
NBUF = 5
AHEAD = 3


def _sc_body(tok_hbm, ids_hbm, pos_hbm, out_hbm,
             idx_all, pos_v,
             buf0, buf1, buf2, buf3, buf4,
             g0, g1, g2, g3, g4, s0, s1, s2, s3, s4):
    bufs = (buf0, buf1, buf2, buf3, buf4)
    gsems = (g0, g1, g2, g3, g4)
    ssems = (s0, s1, s2, s3, s4)

    wid = lax.axis_index("s") * NUM_CORES + lax.axis_index("c")
    b0 = wid * CHUNK

    pltpu.sync_copy(ids_hbm.at[:, pl.ds(b0, CHUNK)], idx_all)
    pltpu.sync_copy(pos_hbm, pos_v)

    for k in range(AHEAD):
        pltpu.async_copy(tok_hbm.at[idx_all.at[k]], bufs[k], gsems[k])

    def process(l, k):
        buf, gsem, ssem = bufs[k], gsems[k], ssems[k]
        pltpu.make_async_copy(tok_hbm.at[pl.ds(0, CHUNK)], buf, gsem).wait()

        pvecs = [pos_v[l, pl.ds(j * LANES, LANES)] for j in range(D // LANES)]

        def radd(r, carry):
            for j in range(D // LANES):
                plsc.addupdate(buf.at[r, pl.ds(j * LANES, LANES)], pvecs[j])
            return carry

        lax.fori_loop(0, CHUNK, radd, 0, unroll=8)

        pltpu.async_copy(buf, out_hbm.at[pl.ds(b0, CHUNK), l, pl.ds(0, D)],
                         ssem)

        ka = (k + AHEAD) % NBUF
        la = l + AHEAD

        @pl.when(la < L)
        def _():
            @pl.when(la >= NBUF)
            def _():
                pltpu.make_async_copy(
                    bufs[ka], out_hbm.at[pl.ds(b0, CHUNK), 0, pl.ds(0, D)],
                    ssems[ka]).wait()
            pltpu.async_copy(tok_hbm.at[idx_all.at[la]], bufs[ka], gsems[ka])

    def ring_lap(i, carry):
        l = NBUF * i
        for k in range(NBUF):
            process(l + k, k)
        return carry

    lax.fori_loop(0, L // NBUF, ring_lap, 0)

    for k in range(NBUF):
        pltpu.make_async_copy(bufs[k],
                              out_hbm.at[pl.ds(b0, CHUNK), 0, pl.ds(0, D)],
                              ssems[k]).wait()


def _make_sc_kernel():
    mesh = plsc.VectorSubcoreMesh(core_axis_name="c", subcore_axis_name="s")
    return pl.kernel(
        _sc_body,
        out_type=jax.ShapeDtypeStruct((B, L, 2 * D), jnp.float32),
        mesh=mesh,
        scratch_types=[
            pltpu.VMEM((L, CHUNK), jnp.int32),
            pltpu.VMEM((L, D), jnp.float32),
        ] + [pltpu.VMEM((CHUNK, D), jnp.float32)] * NBUF
          + [pltpu.SemaphoreType.DMA] * (2 * NBUF),
        compiler_params=pltpu.CompilerParams(use_tc_tiling_on_sc=False),
    )


def _kernel_impl(ids, tok, pos):
    ids_t = ids.astype(jnp.int32).T
    pos2d = pos.reshape(L, D).astype(jnp.float32)
    tok_a = lax.optimization_barrier(tok.reshape(500000, 128))
    tok_b = tok_a.reshape(1000000, 64)
    out_p = _make_sc_kernel()(tok_b, ids_t, pos2d)
    return out_p[:, :, :D]


kernel = jax.jit(_kernel_impl)
